# Initial kernel scaffold; baseline (speedup 1.0000x reference)
import jax
import jax.numpy as jnp
from jax.experimental import pallas as pl


def kernel(x, edge_index, edge_attr, batch, W0, b0, We1, be1, We2, be2, bconv, Wih, Whh, bih, bhh, Ws_ih, Ws_hh, bs_ih, bs_hh):
    raise NotImplementedError("write your pallas kernel here")


# speedup vs baseline: 1.0050x; 1.0050x over previous
import functools

import jax
import jax.numpy as jnp
from jax import lax
from jax.experimental import pallas as pl
from jax.experimental.pallas import tpu as pltpu
from jax.experimental.pallas import tpu_sc as plsc

N = 10000
E = 160000
DIM = 32
B = 128
FIN = 128
W = 128

NC = 2
NS = 16
NW = NC * NS
PER_W = E // NW
CH = 200
NCHUNK = PER_W // CH
ZROWS = 632
NPAD = ZROWS * NS


def _sc_mesh():
    return dict(mesh=plsc.VectorSubcoreMesh(core_axis_name="c",
                                            subcore_axis_name="s",
                                            num_cores=NC, num_subcores=NS))


def _sc_gather(table, src):
    @functools.partial(
        pl.kernel,
        out_type=jax.ShapeDtypeStruct((E, W), jnp.float32),
        scratch_types=[pltpu.VMEM((CH,), jnp.int32),
                       pltpu.VMEM((CH, W), jnp.float32),
                       pltpu.SemaphoreType.DMA],
        **_sc_mesh(),
    )
    def k(table_h, src_h, xj_h, idx_v, rows_v, sem):
        wid = lax.axis_index("c") * NS + lax.axis_index("s")
        base_w = pl.multiple_of(wid * PER_W, 8)

        def chunk(ci, carry):
            base = pl.multiple_of(base_w + ci * CH, 8)
            pltpu.sync_copy(src_h.at[pl.ds(base, CH)], idx_v)
            pltpu.async_copy(table_h.at[idx_v], rows_v, sem).wait()
            pltpu.sync_copy(rows_v, xj_h.at[pl.ds(base, CH)])
            return carry

        lax.fori_loop(0, NCHUNK, chunk, 0)

    return k(table, src)


def _sc_scatter_sum(msg, dst, zrows):
    @functools.partial(
        pl.kernel,
        out_type=[jax.ShapeDtypeStruct((NPAD, W), jnp.float32),
                  jax.ShapeDtypeStruct((NPAD, W), jnp.float32)],
        scratch_types=[pltpu.VMEM((CH,), jnp.int32),
                       pltpu.VMEM((CH, W), jnp.float32),
                       pltpu.VMEM_SHARED((NPAD, W), jnp.float32)],
        **_sc_mesh(),
    )
    def k(msg_h, dst_h, zeros_h, out0_h, out1_h, idx_v, rows_v, acc_s):
        c = lax.axis_index("c")
        s = lax.axis_index("s")
        wid = c * NS + s
        base_w = pl.multiple_of(wid * PER_W, 8)
        zb = pl.multiple_of(s * ZROWS, 8)
        pltpu.sync_copy(zeros_h.at[pl.ds(zb, ZROWS)], acc_s.at[pl.ds(zb, ZROWS)])
        plsc.subcore_barrier()

        def chunk(ci, carry):
            base = pl.multiple_of(base_w + ci * CH, 8)
            pltpu.sync_copy(dst_h.at[pl.ds(base, CH)], idx_v)
            pltpu.sync_copy(msg_h.at[pl.ds(base, CH)], rows_v)
            pltpu.sync_copy(rows_v, acc_s.at[idx_v], add=True)
            return carry

        lax.fori_loop(0, NCHUNK, chunk, 0)
        plsc.subcore_barrier()

        @pl.when(c == 0)
        def _():
            pltpu.sync_copy(acc_s.at[pl.ds(zb, ZROWS)], out0_h.at[pl.ds(zb, ZROWS)])

        @pl.when(c == 1)
        def _():
            pltpu.sync_copy(acc_s.at[pl.ds(zb, ZROWS)], out1_h.at[pl.ds(zb, ZROWS)])

    return k(msg, dst, zrows)


def _sigmoid(x):
    return 1.0 / (1.0 + jnp.exp(-x))


def _tc_proj(x, W0T, b0):
    T = 2000

    def body(x_ref, w_ref, b_ref, o_ref):
        h = jnp.maximum(
            jnp.dot(x_ref[...], w_ref[...],
                    preferred_element_type=jnp.float32) + b_ref[...], 0.0)
        o_ref[...] = jnp.concatenate(
            [h, jnp.zeros((T, W - DIM), jnp.float32)], axis=1)

    return pl.pallas_call(
        body,
        grid=(N // T,),
        in_specs=[pl.BlockSpec((T, FIN), lambda i: (i, 0)),
                  pl.BlockSpec((FIN, DIM), lambda i: (0, 0)),
                  pl.BlockSpec((1, DIM), lambda i: (0, 0))],
        out_specs=pl.BlockSpec((T, W), lambda i: (i, 0)),
        out_shape=jax.ShapeDtypeStruct((N, W), jnp.float32),
    )(x, W0T, b0.reshape(1, DIM))


def _tc_hid(attrP, We1TP, be1):
    T = 2000

    def body(a_ref, w_ref, b_ref, o_ref):
        o_ref[...] = jnp.maximum(
            jnp.dot(a_ref[...], w_ref[...],
                    preferred_element_type=jnp.float32) + b_ref[...], 0.0)

    return pl.pallas_call(
        body,
        grid=(E // T,),
        in_specs=[pl.BlockSpec((T, 8), lambda i: (i, 0)),
                  pl.BlockSpec((8, 128), lambda i: (0, 0)),
                  pl.BlockSpec((1, 128), lambda i: (0, 0))],
        out_specs=pl.BlockSpec((T, 128), lambda i: (i, 0)),
        out_shape=jax.ShapeDtypeStruct((E, 128), jnp.float32),
    )(attrP, We1TP, be1.reshape(1, 128))


def _tc_msg(hid, xj, We2T, be2):
    T = 800

    def body(h_ref, x_ref, w_ref, b_ref, o_ref):
        ew = jnp.dot(h_ref[...], w_ref[...],
                     preferred_element_type=jnp.float32) + b_ref[...]
        xj_t = x_ref[...]
        acc = xj_t[:, 0:1] * ew[:, 0:DIM]
        for d in range(1, DIM):
            acc = acc + xj_t[:, d:d + 1] * ew[:, d * DIM:(d + 1) * DIM]
        o_ref[...] = jnp.concatenate(
            [acc, jnp.ones((T, 1), jnp.float32),
             jnp.zeros((T, W - DIM - 1), jnp.float32)], axis=1)

    return pl.pallas_call(
        body,
        grid=(E // T,),
        in_specs=[pl.BlockSpec((T, 128), lambda i: (i, 0)),
                  pl.BlockSpec((T, W), lambda i: (i, 0)),
                  pl.BlockSpec((128, DIM * DIM), lambda i: (0, 0)),
                  pl.BlockSpec((1, DIM * DIM), lambda i: (0, 0))],
        out_specs=pl.BlockSpec((T, W), lambda i: (i, 0)),
        out_shape=jax.ShapeDtypeStruct((E, W), jnp.float32),
    )(hid, xj, We2T, be2.reshape(1, DIM * DIM))


def _tc_mean_gru(s0, s1, h, WihT, WhhT, bih, bhh, bconv):
    def body(s0_ref, s1_ref, h_ref, wi_ref, wh_ref,
             bi_ref, bh_ref, bc_ref, o_ref):
        ssum = s0_ref[0:N, 0:DIM] + s1_ref[0:N, 0:DIM]
        cnt = s0_ref[0:N, DIM:DIM + 1] + s1_ref[0:N, DIM:DIM + 1]
        aggr = ssum / jnp.maximum(cnt, 1.0) + bc_ref[...]
        m = jnp.maximum(aggr, 0.0)
        h = h_ref[:, 0:DIM]
        gi = jnp.dot(m, wi_ref[...], preferred_element_type=jnp.float32) + bi_ref[...]
        gh = jnp.dot(h, wh_ref[...], preferred_element_type=jnp.float32) + bh_ref[...]
        r = _sigmoid(gi[:, 0:DIM] + gh[:, 0:DIM])
        z = _sigmoid(gi[:, DIM:2 * DIM] + gh[:, DIM:2 * DIM])
        n = jnp.tanh(gi[:, 2 * DIM:] + r * gh[:, 2 * DIM:])
        hn = (1.0 - z) * n + z * h
        o_ref[...] = jnp.concatenate(
            [hn, jnp.zeros((N, W - DIM), jnp.float32)], axis=1)

    return pl.pallas_call(
        body,
        out_shape=jax.ShapeDtypeStruct((N, W), jnp.float32),
    )(s0, s1, h, WihT, WhhT,
      bih.reshape(1, 3 * DIM), bhh.reshape(1, 3 * DIM), bconv.reshape(1, DIM))


def _tc_set2set(out, batch2, WsihT, WshhT, bsih, bshh):
    def body(out_ref, b_ref, wi_ref, wh_ref, bi_ref, bh_ref, q_ref, f_ref):
        feat = out_ref[:, 0:DIM]
        f_ref[...] = feat
        seg = b_ref[...]
        oh = (seg == lax.broadcasted_iota(jnp.int32, (N, B), 1)
              ).astype(jnp.float32)
        q_star = jnp.zeros((B, 2 * DIM), jnp.float32)
        hs = jnp.zeros((B, DIM), jnp.float32)
        cs = jnp.zeros((B, DIM), jnp.float32)
        dn = (((0,), (0,)), ((), ()))
        for _ in range(3):
            g = (jnp.dot(q_star, wi_ref[...], preferred_element_type=jnp.float32)
                 + bi_ref[...]
                 + jnp.dot(hs, wh_ref[...], preferred_element_type=jnp.float32)
                 + bh_ref[...])
            i = _sigmoid(g[:, 0:DIM])
            f = _sigmoid(g[:, DIM:2 * DIM])
            gg = jnp.tanh(g[:, 2 * DIM:3 * DIM])
            o = _sigmoid(g[:, 3 * DIM:])
            cs = f * cs + i * gg
            hs = o * jnp.tanh(cs)
            qn = jnp.dot(oh, hs, preferred_element_type=jnp.float32)
            e = jnp.sum(feat * qn, axis=1, keepdims=True)
            emax = jnp.max(jnp.where(oh > 0.0, e, -1e30), axis=0,
                           keepdims=True)
            emaxn = jnp.dot(oh, emax.reshape(B, 1),
                            preferred_element_type=jnp.float32)
            a = jnp.exp(e - emaxn)
            denom = lax.dot_general(oh, a, dn,
                                    preferred_element_type=jnp.float32)
            denomn = jnp.dot(oh, denom, preferred_element_type=jnp.float32)
            a = a / denomn
            r = lax.dot_general(oh, a * feat, dn,
                                preferred_element_type=jnp.float32)
            q_star = jnp.concatenate([hs, r], axis=1)
        q_ref[...] = q_star

    return pl.pallas_call(
        body,
        out_shape=[jax.ShapeDtypeStruct((B, 2 * DIM), jnp.float32),
                   jax.ShapeDtypeStruct((N, DIM), jnp.float32)],
    )(out, batch2, WsihT, WshhT,
      bsih.reshape(1, 4 * DIM), bshh.reshape(1, 4 * DIM))


def kernel(x, edge_index, edge_attr, batch, W0, b0, We1, be1, We2, be2,
           bconv, Wih, Whh, bih, bhh, Ws_ih, Ws_hh, bs_ih, bs_hh):
    src = edge_index[0]
    dst = edge_index[1]
    attrP = jnp.pad(edge_attr, ((0, 0), (0, 3)))
    We1TP = jnp.pad(We1.T, ((0, 3), (0, 0)))
    zrows = jnp.zeros((NPAD, W), jnp.float32)
    batch2 = batch.reshape(N, 1)

    out = _tc_proj(x, W0.T, b0)
    hid = _tc_hid(attrP, We1TP, be1)

    We2T = We2.T
    WihT = Wih.T
    WhhT = Whh.T
    for _ in range(3):
        xj = _sc_gather(out, src)
        msg = _tc_msg(hid, xj, We2T, be2)
        s0, s1 = _sc_scatter_sum(msg, dst, zrows)
        out = _tc_mean_gru(s0, s1, out, WihT, WhhT, bih, bhh, bconv)

    q_star, feat = _tc_set2set(out, batch2, Ws_ih.T, Ws_hh.T, bs_ih, bs_hh)
    return (q_star, feat)


# speedup vs baseline: 2.5551x; 2.5423x over previous
import functools

import jax
import jax.numpy as jnp
from jax import lax
from jax.experimental import pallas as pl
from jax.experimental.pallas import tpu as pltpu
from jax.experimental.pallas import tpu_sc as plsc

N = 10000
E = 160000
DIM = 32
B = 128
FIN = 128
W = 128

NC = 2
NS = 16
NW = NC * NS
PER_W = E // NW
CH = 200
NCHUNK = PER_W // CH
ZROWS = 632
NPAD = ZROWS * NS


def _sc_mesh():
    return dict(mesh=plsc.VectorSubcoreMesh(core_axis_name="c",
                                            subcore_axis_name="s",
                                            num_cores=NC, num_subcores=NS))


def _sc_gather(table, src):
    @functools.partial(
        pl.kernel,
        out_type=jax.ShapeDtypeStruct((E, W), jnp.float32),
        scratch_types=[pltpu.VMEM((CH,), jnp.int32),
                       pltpu.VMEM((CH, W), jnp.float32),
                       pltpu.SemaphoreType.DMA],
        **_sc_mesh(),
    )
    def k(table_h, src_h, xj_h, idx_v, rows_v, sem):
        wid = lax.axis_index("c") * NS + lax.axis_index("s")
        base_w = pl.multiple_of(wid * PER_W, 8)

        def chunk(ci, carry):
            base = pl.multiple_of(base_w + ci * CH, 8)
            pltpu.sync_copy(src_h.at[pl.ds(base, CH)], idx_v)
            pltpu.async_copy(table_h.at[idx_v], rows_v, sem).wait()
            pltpu.sync_copy(rows_v, xj_h.at[pl.ds(base, CH)])
            return carry

        lax.fori_loop(0, NCHUNK, chunk, 0)

    return k(table, src)


def _sc_scatter_sum(msg, dst, zrows):
    @functools.partial(
        pl.kernel,
        out_type=[jax.ShapeDtypeStruct((NPAD, W), jnp.float32),
                  jax.ShapeDtypeStruct((NPAD, W), jnp.float32)],
        scratch_types=[pltpu.VMEM((CH,), jnp.int32),
                       pltpu.VMEM((CH, W), jnp.float32),
                       pltpu.VMEM_SHARED((NPAD, W), jnp.float32)],
        **_sc_mesh(),
    )
    def k(msg_h, dst_h, zeros_h, out0_h, out1_h, idx_v, rows_v, acc_s):
        c = lax.axis_index("c")
        s = lax.axis_index("s")
        wid = c * NS + s
        base_w = pl.multiple_of(wid * PER_W, 8)
        zb = pl.multiple_of(s * ZROWS, 8)
        pltpu.sync_copy(zeros_h.at[pl.ds(zb, ZROWS)], acc_s.at[pl.ds(zb, ZROWS)])
        plsc.subcore_barrier()

        def chunk(ci, carry):
            base = pl.multiple_of(base_w + ci * CH, 8)
            pltpu.sync_copy(dst_h.at[pl.ds(base, CH)], idx_v)
            pltpu.sync_copy(msg_h.at[pl.ds(base, CH)], rows_v)
            pltpu.sync_copy(rows_v, acc_s.at[idx_v], add=True)
            return carry

        lax.fori_loop(0, NCHUNK, chunk, 0)
        plsc.subcore_barrier()

        @pl.when(c == 0)
        def _():
            pltpu.sync_copy(acc_s.at[pl.ds(zb, ZROWS)], out0_h.at[pl.ds(zb, ZROWS)])

        @pl.when(c == 1)
        def _():
            pltpu.sync_copy(acc_s.at[pl.ds(zb, ZROWS)], out1_h.at[pl.ds(zb, ZROWS)])

    return k(msg, dst, zrows)


def _sigmoid(x):
    return 1.0 / (1.0 + jnp.exp(-x))


def _tc_proj(x, W0T, b0):
    T = 2000

    def body(x_ref, w_ref, b_ref, o_ref):
        h = jnp.maximum(
            jnp.dot(x_ref[...], w_ref[...],
                    preferred_element_type=jnp.float32) + b_ref[...], 0.0)
        o_ref[...] = jnp.concatenate(
            [h, jnp.zeros((T, W - DIM), jnp.float32)], axis=1)

    return pl.pallas_call(
        body,
        grid=(N // T,),
        in_specs=[pl.BlockSpec((T, FIN), lambda i: (i, 0)),
                  pl.BlockSpec((FIN, DIM), lambda i: (0, 0)),
                  pl.BlockSpec((1, DIM), lambda i: (0, 0))],
        out_specs=pl.BlockSpec((T, W), lambda i: (i, 0)),
        out_shape=jax.ShapeDtypeStruct((N, W), jnp.float32),
    )(x, W0T, b0.reshape(1, DIM))


def _tc_hid(attrP, We1TP, be1):
    T = 2000

    def body(a_ref, w_ref, b_ref, o_ref):
        o_ref[...] = jnp.maximum(
            jnp.dot(a_ref[...], w_ref[...],
                    preferred_element_type=jnp.float32) + b_ref[...], 0.0)

    return pl.pallas_call(
        body,
        grid=(E // T,),
        in_specs=[pl.BlockSpec((T, 8), lambda i: (i, 0)),
                  pl.BlockSpec((8, 128), lambda i: (0, 0)),
                  pl.BlockSpec((1, 128), lambda i: (0, 0))],
        out_specs=pl.BlockSpec((T, 128), lambda i: (i, 0)),
        out_shape=jax.ShapeDtypeStruct((E, 128), jnp.float32),
    )(attrP, We1TP, be1.reshape(1, 128))


def _tc_msg(hid, xj, We2T, be2, Rrep, Ssum):
    T = 800

    def body(h_ref, x_ref, w_ref, b_ref, r_ref, s_ref, o_ref):
        ew = jnp.dot(h_ref[...], w_ref[...],
                     preferred_element_type=jnp.float32) + b_ref[...]
        xjb = jnp.dot(x_ref[:, 0:DIM], r_ref[...],
                      preferred_element_type=jnp.float32)
        acc = jnp.dot(xjb * ew, s_ref[...],
                      preferred_element_type=jnp.float32)
        o_ref[...] = jnp.concatenate(
            [acc, jnp.ones((T, 1), jnp.float32),
             jnp.zeros((T, W - DIM - 1), jnp.float32)], axis=1)

    return pl.pallas_call(
        body,
        grid=(E // T,),
        in_specs=[pl.BlockSpec((T, 128), lambda i: (i, 0)),
                  pl.BlockSpec((T, W), lambda i: (i, 0)),
                  pl.BlockSpec((128, DIM * DIM), lambda i: (0, 0)),
                  pl.BlockSpec((1, DIM * DIM), lambda i: (0, 0)),
                  pl.BlockSpec((DIM, DIM * DIM), lambda i: (0, 0)),
                  pl.BlockSpec((DIM * DIM, DIM), lambda i: (0, 0))],
        out_specs=pl.BlockSpec((T, W), lambda i: (i, 0)),
        out_shape=jax.ShapeDtypeStruct((E, W), jnp.float32),
    )(hid, xj, We2T, be2.reshape(1, DIM * DIM), Rrep, Ssum)


def _tc_mean_gru(s0, s1, h, WihT, WhhT, bih, bhh, bconv):
    def body(s0_ref, s1_ref, h_ref, wi_ref, wh_ref,
             bi_ref, bh_ref, bc_ref, o_ref):
        ssum = s0_ref[0:N, 0:DIM] + s1_ref[0:N, 0:DIM]
        cnt = s0_ref[0:N, DIM:DIM + 1] + s1_ref[0:N, DIM:DIM + 1]
        aggr = ssum / jnp.maximum(cnt, 1.0) + bc_ref[...]
        m = jnp.maximum(aggr, 0.0)
        h = h_ref[:, 0:DIM]
        gi = jnp.dot(m, wi_ref[...], preferred_element_type=jnp.float32) + bi_ref[...]
        gh = jnp.dot(h, wh_ref[...], preferred_element_type=jnp.float32) + bh_ref[...]
        r = _sigmoid(gi[:, 0:DIM] + gh[:, 0:DIM])
        z = _sigmoid(gi[:, DIM:2 * DIM] + gh[:, DIM:2 * DIM])
        n = jnp.tanh(gi[:, 2 * DIM:] + r * gh[:, 2 * DIM:])
        hn = (1.0 - z) * n + z * h
        o_ref[...] = jnp.concatenate(
            [hn, jnp.zeros((N, W - DIM), jnp.float32)], axis=1)

    return pl.pallas_call(
        body,
        out_shape=jax.ShapeDtypeStruct((N, W), jnp.float32),
    )(s0, s1, h, WihT, WhhT,
      bih.reshape(1, 3 * DIM), bhh.reshape(1, 3 * DIM), bconv.reshape(1, DIM))


def _tc_set2set(out, batch2, WsihT, WshhT, bsih, bshh):
    def body(out_ref, b_ref, wi_ref, wh_ref, bi_ref, bh_ref, q_ref, f_ref):
        feat = out_ref[:, 0:DIM]
        f_ref[...] = feat
        seg = b_ref[...]
        oh = (seg == lax.broadcasted_iota(jnp.int32, (N, B), 1)
              ).astype(jnp.float32)
        q_star = jnp.zeros((B, 2 * DIM), jnp.float32)
        hs = jnp.zeros((B, DIM), jnp.float32)
        cs = jnp.zeros((B, DIM), jnp.float32)
        dn = (((0,), (0,)), ((), ()))
        for _ in range(3):
            g = (jnp.dot(q_star, wi_ref[...], preferred_element_type=jnp.float32)
                 + bi_ref[...]
                 + jnp.dot(hs, wh_ref[...], preferred_element_type=jnp.float32)
                 + bh_ref[...])
            i = _sigmoid(g[:, 0:DIM])
            f = _sigmoid(g[:, DIM:2 * DIM])
            gg = jnp.tanh(g[:, 2 * DIM:3 * DIM])
            o = _sigmoid(g[:, 3 * DIM:])
            cs = f * cs + i * gg
            hs = o * jnp.tanh(cs)
            qn = jnp.dot(oh, hs, preferred_element_type=jnp.float32)
            e = jnp.sum(feat * qn, axis=1, keepdims=True)
            emax = jnp.max(jnp.where(oh > 0.0, e, -1e30), axis=0,
                           keepdims=True)
            emaxn = jnp.dot(oh, emax.reshape(B, 1),
                            preferred_element_type=jnp.float32)
            a = jnp.exp(e - emaxn)
            denom = lax.dot_general(oh, a, dn,
                                    preferred_element_type=jnp.float32)
            denomn = jnp.dot(oh, denom, preferred_element_type=jnp.float32)
            a = a / denomn
            r = lax.dot_general(oh, a * feat, dn,
                                preferred_element_type=jnp.float32)
            q_star = jnp.concatenate([hs, r], axis=1)
        q_ref[...] = q_star

    return pl.pallas_call(
        body,
        out_shape=[jax.ShapeDtypeStruct((B, 2 * DIM), jnp.float32),
                   jax.ShapeDtypeStruct((N, DIM), jnp.float32)],
    )(out, batch2, WsihT, WshhT,
      bsih.reshape(1, 4 * DIM), bshh.reshape(1, 4 * DIM))


def kernel(x, edge_index, edge_attr, batch, W0, b0, We1, be1, We2, be2,
           bconv, Wih, Whh, bih, bhh, Ws_ih, Ws_hh, bs_ih, bs_hh):
    src = edge_index[0]
    dst = edge_index[1]
    attrP = jnp.pad(edge_attr, ((0, 0), (0, 3)))
    We1TP = jnp.pad(We1.T, ((0, 3), (0, 0)))
    zrows = jnp.zeros((NPAD, W), jnp.float32)
    batch2 = batch.reshape(N, 1)

    out = _tc_proj(x, W0.T, b0)
    hid = _tc_hid(attrP, We1TP, be1)

    We2T = We2.T
    WihT = Wih.T
    WhhT = Whh.T
    Rrep = jnp.repeat(jnp.eye(DIM, dtype=jnp.float32), DIM, axis=1)
    Ssum = jnp.tile(jnp.eye(DIM, dtype=jnp.float32), (DIM, 1))
    for _ in range(3):
        xj = _sc_gather(out, src)
        msg = _tc_msg(hid, xj, We2T, be2, Rrep, Ssum)
        s0, s1 = _sc_scatter_sum(msg, dst, zrows)
        out = _tc_mean_gru(s0, s1, out, WihT, WhhT, bih, bhh, bconv)

    q_star, feat = _tc_set2set(out, batch2, Ws_ih.T, Ws_hh.T, bs_ih, bs_hh)
    return (q_star, feat)


# speedup vs baseline: 2.6321x; 1.0301x over previous
import functools

import jax
import jax.numpy as jnp
from jax import lax
from jax.experimental import pallas as pl
from jax.experimental.pallas import tpu as pltpu
from jax.experimental.pallas import tpu_sc as plsc

N = 10000
E = 160000
DIM = 32
B = 128
FIN = 128
W = 128

NC = 2
NS = 16
NW = NC * NS
PER_W = E // NW
CH = 200
NCHUNK = PER_W // CH
ZROWS = 632
NPAD = ZROWS * NS


def _sc_mesh():
    return dict(mesh=plsc.VectorSubcoreMesh(core_axis_name="c",
                                            subcore_axis_name="s",
                                            num_cores=NC, num_subcores=NS))


def _sc_gather(table, src):
    @functools.partial(
        pl.kernel,
        out_type=jax.ShapeDtypeStruct((E, W), jnp.float32),
        scratch_types=[pltpu.VMEM((PER_W,), jnp.int32),
                       pltpu.VMEM((CH, W), jnp.float32),
                       pltpu.VMEM((CH, W), jnp.float32),
                       pltpu.SemaphoreType.DMA,
                       pltpu.SemaphoreType.DMA,
                       pltpu.SemaphoreType.DMA,
                       pltpu.SemaphoreType.DMA],
        **_sc_mesh(),
    )
    def k(table_h, src_h, xj_h, idx_all, rows0, rows1, sg0, sg1, sw0, sw1):
        wid = lax.axis_index("c") * NS + lax.axis_index("s")
        base_w = pl.multiple_of(wid * PER_W, 8)
        pltpu.sync_copy(src_h.at[pl.ds(base_w, PER_W)], idx_all)
        rows = (rows0, rows1)
        sg = (sg0, sg1)
        sw = (sw0, sw1)
        hg = [None, None]
        hw = [None, None]
        for ci in range(NCHUNK):
            b = ci & 1
            if hw[b] is not None:
                hw[b].wait()
                hw[b] = None
            hg[b] = pltpu.async_copy(
                table_h.at[idx_all.at[pl.ds(ci * CH, CH)]], rows[b], sg[b])
            if ci > 0:
                pb = 1 - b
                hg[pb].wait()
                hw[pb] = pltpu.async_copy(
                    rows[pb],
                    xj_h.at[pl.ds(base_w + (ci - 1) * CH, CH)], sw[pb])
        bl = (NCHUNK - 1) & 1
        hg[bl].wait()
        hw[bl] = pltpu.async_copy(
            rows[bl], xj_h.at[pl.ds(base_w + (NCHUNK - 1) * CH, CH)], sw[bl])
        for b in range(2):
            if hw[b] is not None:
                hw[b].wait()

    return k(table, src)


def _sc_scatter_sum(msg, dst, zrows):
    @functools.partial(
        pl.kernel,
        out_type=[jax.ShapeDtypeStruct((NPAD, W), jnp.float32),
                  jax.ShapeDtypeStruct((NPAD, W), jnp.float32)],
        scratch_types=[pltpu.VMEM((CH,), jnp.int32),
                       pltpu.VMEM((CH, W), jnp.float32),
                       pltpu.VMEM_SHARED((NPAD, W), jnp.float32)],
        **_sc_mesh(),
    )
    def k(msg_h, dst_h, zeros_h, out0_h, out1_h, idx_v, rows_v, acc_s):
        c = lax.axis_index("c")
        s = lax.axis_index("s")
        wid = c * NS + s
        base_w = pl.multiple_of(wid * PER_W, 8)
        zb = pl.multiple_of(s * ZROWS, 8)
        pltpu.sync_copy(zeros_h.at[pl.ds(zb, ZROWS)], acc_s.at[pl.ds(zb, ZROWS)])
        plsc.subcore_barrier()

        def chunk(ci, carry):
            base = pl.multiple_of(base_w + ci * CH, 8)
            pltpu.sync_copy(dst_h.at[pl.ds(base, CH)], idx_v)
            pltpu.sync_copy(msg_h.at[pl.ds(base, CH)], rows_v)
            pltpu.sync_copy(rows_v, acc_s.at[idx_v], add=True)
            return carry

        lax.fori_loop(0, NCHUNK, chunk, 0)
        plsc.subcore_barrier()

        @pl.when(c == 0)
        def _():
            pltpu.sync_copy(acc_s.at[pl.ds(zb, ZROWS)], out0_h.at[pl.ds(zb, ZROWS)])

        @pl.when(c == 1)
        def _():
            pltpu.sync_copy(acc_s.at[pl.ds(zb, ZROWS)], out1_h.at[pl.ds(zb, ZROWS)])

    return k(msg, dst, zrows)


def _sigmoid(x):
    return 1.0 / (1.0 + jnp.exp(-x))


def _tc_proj(x, W0T, b0):
    T = 2000

    def body(x_ref, w_ref, b_ref, o_ref):
        h = jnp.maximum(
            jnp.dot(x_ref[...], w_ref[...],
                    preferred_element_type=jnp.float32) + b_ref[...], 0.0)
        o_ref[...] = jnp.concatenate(
            [h, jnp.zeros((T, W - DIM), jnp.float32)], axis=1)

    return pl.pallas_call(
        body,
        grid=(N // T,),
        in_specs=[pl.BlockSpec((T, FIN), lambda i: (i, 0)),
                  pl.BlockSpec((FIN, DIM), lambda i: (0, 0)),
                  pl.BlockSpec((1, DIM), lambda i: (0, 0))],
        out_specs=pl.BlockSpec((T, W), lambda i: (i, 0)),
        out_shape=jax.ShapeDtypeStruct((N, W), jnp.float32),
    )(x, W0T, b0.reshape(1, DIM))


def _tc_hid(attrP, We1TP, be1):
    T = 2000

    def body(a_ref, w_ref, b_ref, o_ref):
        o_ref[...] = jnp.maximum(
            jnp.dot(a_ref[...], w_ref[...],
                    preferred_element_type=jnp.float32) + b_ref[...], 0.0)

    return pl.pallas_call(
        body,
        grid=(E // T,),
        in_specs=[pl.BlockSpec((T, 8), lambda i: (i, 0)),
                  pl.BlockSpec((8, 128), lambda i: (0, 0)),
                  pl.BlockSpec((1, 128), lambda i: (0, 0))],
        out_specs=pl.BlockSpec((T, 128), lambda i: (i, 0)),
        out_shape=jax.ShapeDtypeStruct((E, 128), jnp.float32),
    )(attrP, We1TP, be1.reshape(1, 128))


def _tc_msg(hid, xj, We2T, be2, Rrep, Ssum):
    T = 800

    def body(h_ref, x_ref, w_ref, b_ref, r_ref, s_ref, o_ref):
        ew = jnp.dot(h_ref[...], w_ref[...],
                     preferred_element_type=jnp.float32) + b_ref[...]
        xjb = jnp.dot(x_ref[:, 0:DIM], r_ref[...],
                      preferred_element_type=jnp.float32)
        acc = jnp.dot(xjb * ew, s_ref[...],
                      preferred_element_type=jnp.float32)
        o_ref[...] = jnp.concatenate(
            [acc, jnp.ones((T, 1), jnp.float32),
             jnp.zeros((T, W - DIM - 1), jnp.float32)], axis=1)

    return pl.pallas_call(
        body,
        grid=(E // T,),
        in_specs=[pl.BlockSpec((T, 128), lambda i: (i, 0)),
                  pl.BlockSpec((T, W), lambda i: (i, 0)),
                  pl.BlockSpec((128, DIM * DIM), lambda i: (0, 0)),
                  pl.BlockSpec((1, DIM * DIM), lambda i: (0, 0)),
                  pl.BlockSpec((DIM, DIM * DIM), lambda i: (0, 0)),
                  pl.BlockSpec((DIM * DIM, DIM), lambda i: (0, 0))],
        out_specs=pl.BlockSpec((T, W), lambda i: (i, 0)),
        out_shape=jax.ShapeDtypeStruct((E, W), jnp.float32),
    )(hid, xj, We2T, be2.reshape(1, DIM * DIM), Rrep, Ssum)


def _tc_mean_gru(s0, s1, h, WihT, WhhT, bih, bhh, bconv):
    def body(s0_ref, s1_ref, h_ref, wi_ref, wh_ref,
             bi_ref, bh_ref, bc_ref, o_ref):
        ssum = s0_ref[0:N, 0:DIM] + s1_ref[0:N, 0:DIM]
        cnt = s0_ref[0:N, DIM:DIM + 1] + s1_ref[0:N, DIM:DIM + 1]
        aggr = ssum / jnp.maximum(cnt, 1.0) + bc_ref[...]
        m = jnp.maximum(aggr, 0.0)
        h = h_ref[:, 0:DIM]
        gi = jnp.dot(m, wi_ref[...], preferred_element_type=jnp.float32) + bi_ref[...]
        gh = jnp.dot(h, wh_ref[...], preferred_element_type=jnp.float32) + bh_ref[...]
        r = _sigmoid(gi[:, 0:DIM] + gh[:, 0:DIM])
        z = _sigmoid(gi[:, DIM:2 * DIM] + gh[:, DIM:2 * DIM])
        n = jnp.tanh(gi[:, 2 * DIM:] + r * gh[:, 2 * DIM:])
        hn = (1.0 - z) * n + z * h
        o_ref[...] = jnp.concatenate(
            [hn, jnp.zeros((N, W - DIM), jnp.float32)], axis=1)

    return pl.pallas_call(
        body,
        out_shape=jax.ShapeDtypeStruct((N, W), jnp.float32),
    )(s0, s1, h, WihT, WhhT,
      bih.reshape(1, 3 * DIM), bhh.reshape(1, 3 * DIM), bconv.reshape(1, DIM))


def _tc_set2set(out, batch2, WsihT, WshhT, bsih, bshh):
    def body(out_ref, b_ref, wi_ref, wh_ref, bi_ref, bh_ref, q_ref, f_ref):
        feat = out_ref[:, 0:DIM]
        f_ref[...] = feat
        seg = b_ref[...]
        oh = (seg == lax.broadcasted_iota(jnp.int32, (N, B), 1)
              ).astype(jnp.float32)
        q_star = jnp.zeros((B, 2 * DIM), jnp.float32)
        hs = jnp.zeros((B, DIM), jnp.float32)
        cs = jnp.zeros((B, DIM), jnp.float32)
        dn = (((0,), (0,)), ((), ()))
        for _ in range(3):
            g = (jnp.dot(q_star, wi_ref[...], preferred_element_type=jnp.float32)
                 + bi_ref[...]
                 + jnp.dot(hs, wh_ref[...], preferred_element_type=jnp.float32)
                 + bh_ref[...])
            i = _sigmoid(g[:, 0:DIM])
            f = _sigmoid(g[:, DIM:2 * DIM])
            gg = jnp.tanh(g[:, 2 * DIM:3 * DIM])
            o = _sigmoid(g[:, 3 * DIM:])
            cs = f * cs + i * gg
            hs = o * jnp.tanh(cs)
            qn = jnp.dot(oh, hs, preferred_element_type=jnp.float32)
            e = jnp.sum(feat * qn, axis=1, keepdims=True)
            emax = jnp.max(jnp.where(oh > 0.0, e, -1e30), axis=0,
                           keepdims=True)
            emaxn = jnp.dot(oh, emax.reshape(B, 1),
                            preferred_element_type=jnp.float32)
            a = jnp.exp(e - emaxn)
            denom = lax.dot_general(oh, a, dn,
                                    preferred_element_type=jnp.float32)
            denomn = jnp.dot(oh, denom, preferred_element_type=jnp.float32)
            a = a / denomn
            r = lax.dot_general(oh, a * feat, dn,
                                preferred_element_type=jnp.float32)
            q_star = jnp.concatenate([hs, r], axis=1)
        q_ref[...] = q_star

    return pl.pallas_call(
        body,
        out_shape=[jax.ShapeDtypeStruct((B, 2 * DIM), jnp.float32),
                   jax.ShapeDtypeStruct((N, DIM), jnp.float32)],
    )(out, batch2, WsihT, WshhT,
      bsih.reshape(1, 4 * DIM), bshh.reshape(1, 4 * DIM))


def kernel(x, edge_index, edge_attr, batch, W0, b0, We1, be1, We2, be2,
           bconv, Wih, Whh, bih, bhh, Ws_ih, Ws_hh, bs_ih, bs_hh):
    src = edge_index[0]
    dst = edge_index[1]
    attrP = jnp.pad(edge_attr, ((0, 0), (0, 3)))
    We1TP = jnp.pad(We1.T, ((0, 3), (0, 0)))
    zrows = jnp.zeros((NPAD, W), jnp.float32)
    batch2 = batch.reshape(N, 1)

    out = _tc_proj(x, W0.T, b0)
    hid = _tc_hid(attrP, We1TP, be1)

    We2T = We2.T
    WihT = Wih.T
    WhhT = Whh.T
    Rrep = jnp.repeat(jnp.eye(DIM, dtype=jnp.float32), DIM, axis=1)
    Ssum = jnp.tile(jnp.eye(DIM, dtype=jnp.float32), (DIM, 1))
    for _ in range(3):
        xj = _sc_gather(out, src)
        msg = _tc_msg(hid, xj, We2T, be2, Rrep, Ssum)
        s0, s1 = _sc_scatter_sum(msg, dst, zrows)
        out = _tc_mean_gru(s0, s1, out, WihT, WhhT, bih, bhh, bconv)

    q_star, feat = _tc_set2set(out, batch2, Ws_ih.T, Ws_hh.T, bs_ih, bs_hh)
    return (q_star, feat)


# speedup vs baseline: 2.7452x; 1.0429x over previous
import functools

import jax
import jax.numpy as jnp
from jax import lax
from jax.experimental import pallas as pl
from jax.experimental.pallas import tpu as pltpu
from jax.experimental.pallas import tpu_sc as plsc

N = 10000
E = 160000
DIM = 32
B = 128
FIN = 128
W = 128

NC = 2
NS = 16
NW = NC * NS
PER_W = E // NW
CH = 200
NCHUNK = PER_W // CH
ZROWS = 632
NPAD = ZROWS * NS


def _sc_mesh():
    return dict(mesh=plsc.VectorSubcoreMesh(core_axis_name="c",
                                            subcore_axis_name="s",
                                            num_cores=NC, num_subcores=NS))


def _sc_gather(table, src):
    @functools.partial(
        pl.kernel,
        out_type=jax.ShapeDtypeStruct((E, W), jnp.float32),
        scratch_types=[pltpu.VMEM((PER_W,), jnp.int32),
                       pltpu.VMEM((CH, W), jnp.float32),
                       pltpu.VMEM((CH, W), jnp.float32),
                       pltpu.SemaphoreType.DMA,
                       pltpu.SemaphoreType.DMA,
                       pltpu.SemaphoreType.DMA,
                       pltpu.SemaphoreType.DMA],
        **_sc_mesh(),
    )
    def k(table_h, src_h, xj_h, idx_all, rows0, rows1, sg0, sg1, sw0, sw1):
        wid = lax.axis_index("c") * NS + lax.axis_index("s")
        base_w = pl.multiple_of(wid * PER_W, 8)
        pltpu.sync_copy(src_h.at[pl.ds(base_w, PER_W)], idx_all)
        rows = (rows0, rows1)
        sg = (sg0, sg1)
        sw = (sw0, sw1)
        hg = [None, None]
        hw = [None, None]
        for ci in range(NCHUNK):
            b = ci & 1
            if hw[b] is not None:
                hw[b].wait()
                hw[b] = None
            hg[b] = pltpu.async_copy(
                table_h.at[idx_all.at[pl.ds(ci * CH, CH)]], rows[b], sg[b])
            if ci > 0:
                pb = 1 - b
                hg[pb].wait()
                hw[pb] = pltpu.async_copy(
                    rows[pb],
                    xj_h.at[pl.ds(base_w + (ci - 1) * CH, CH)], sw[pb])
        bl = (NCHUNK - 1) & 1
        hg[bl].wait()
        hw[bl] = pltpu.async_copy(
            rows[bl], xj_h.at[pl.ds(base_w + (NCHUNK - 1) * CH, CH)], sw[bl])
        for b in range(2):
            if hw[b] is not None:
                hw[b].wait()

    return k(table, src)


def _sc_scatter_sum(msg, dst, zrows):
    @functools.partial(
        pl.kernel,
        out_type=[jax.ShapeDtypeStruct((NPAD, W), jnp.float32),
                  jax.ShapeDtypeStruct((NPAD, W), jnp.float32)],
        scratch_types=[pltpu.VMEM((CH,), jnp.int32),
                       pltpu.VMEM((CH, W), jnp.float32),
                       pltpu.VMEM_SHARED((NPAD, W), jnp.float32)],
        **_sc_mesh(),
    )
    def k(msg_h, dst_h, zeros_h, out0_h, out1_h, idx_v, rows_v, acc_s):
        c = lax.axis_index("c")
        s = lax.axis_index("s")
        wid = c * NS + s
        base_w = pl.multiple_of(wid * PER_W, 8)
        zb = pl.multiple_of(s * ZROWS, 8)
        pltpu.sync_copy(zeros_h.at[pl.ds(zb, ZROWS)], acc_s.at[pl.ds(zb, ZROWS)])
        plsc.subcore_barrier()

        def chunk(ci, carry):
            base = pl.multiple_of(base_w + ci * CH, 8)
            pltpu.sync_copy(dst_h.at[pl.ds(base, CH)], idx_v)
            pltpu.sync_copy(msg_h.at[pl.ds(base, CH)], rows_v)
            pltpu.sync_copy(rows_v, acc_s.at[idx_v], add=True)
            return carry

        lax.fori_loop(0, NCHUNK, chunk, 0)
        plsc.subcore_barrier()

        @pl.when(c == 0)
        def _():
            pltpu.sync_copy(acc_s.at[pl.ds(zb, ZROWS)], out0_h.at[pl.ds(zb, ZROWS)])

        @pl.when(c == 1)
        def _():
            pltpu.sync_copy(acc_s.at[pl.ds(zb, ZROWS)], out1_h.at[pl.ds(zb, ZROWS)])

    return k(msg, dst, zrows)


def _sigmoid(x):
    return 1.0 / (1.0 + jnp.exp(-x))


def _tc_proj(x, W0T, b0):
    T = 2000

    def body(x_ref, w_ref, b_ref, o_ref):
        h = jnp.maximum(
            jnp.dot(x_ref[...], w_ref[...],
                    preferred_element_type=jnp.float32) + b_ref[...], 0.0)
        o_ref[...] = jnp.concatenate(
            [h, jnp.zeros((T, W - DIM), jnp.float32)], axis=1)

    return pl.pallas_call(
        body,
        grid=(N // T,),
        in_specs=[pl.BlockSpec((T, FIN), lambda i: (i, 0)),
                  pl.BlockSpec((FIN, DIM), lambda i: (0, 0)),
                  pl.BlockSpec((1, DIM), lambda i: (0, 0))],
        out_specs=pl.BlockSpec((T, W), lambda i: (i, 0)),
        out_shape=jax.ShapeDtypeStruct((N, W), jnp.float32),
    )(x, W0T, b0.reshape(1, DIM))


def _tc_hid(attrP, We1TP, be1):
    T = 2000

    def body(a_ref, w_ref, b_ref, o_ref):
        o_ref[...] = jnp.maximum(
            jnp.dot(a_ref[...], w_ref[...],
                    preferred_element_type=jnp.float32) + b_ref[...], 0.0)

    return pl.pallas_call(
        body,
        grid=(E // T,),
        in_specs=[pl.BlockSpec((T, 8), lambda i: (i, 0)),
                  pl.BlockSpec((8, 128), lambda i: (0, 0)),
                  pl.BlockSpec((1, 128), lambda i: (0, 0))],
        out_specs=pl.BlockSpec((T, 128), lambda i: (i, 0)),
        out_shape=jax.ShapeDtypeStruct((E, 128), jnp.float32),
    )(attrP, We1TP, be1.reshape(1, 128))


def _tc_msg(hid, xj, We2T, be2, Rrep, Ssum):
    T = 1600

    def body(h_ref, x_ref, w_ref, b_ref, r_ref, s_ref, o_ref):
        ew = jnp.dot(h_ref[...].astype(jnp.bfloat16),
                     w_ref[...].astype(jnp.bfloat16),
                     preferred_element_type=jnp.float32) + b_ref[...]
        xjb = jnp.dot(x_ref[:, 0:DIM].astype(jnp.bfloat16),
                      r_ref[...].astype(jnp.bfloat16),
                      preferred_element_type=jnp.float32)
        acc = jnp.dot((xjb * ew).astype(jnp.bfloat16),
                      s_ref[...].astype(jnp.bfloat16),
                      preferred_element_type=jnp.float32)
        o_ref[...] = jnp.concatenate(
            [acc, jnp.ones((T, 1), jnp.float32),
             jnp.zeros((T, W - DIM - 1), jnp.float32)], axis=1)

    return pl.pallas_call(
        body,
        grid=(E // T,),
        in_specs=[pl.BlockSpec((T, 128), lambda i: (i, 0)),
                  pl.BlockSpec((T, W), lambda i: (i, 0)),
                  pl.BlockSpec((128, DIM * DIM), lambda i: (0, 0)),
                  pl.BlockSpec((1, DIM * DIM), lambda i: (0, 0)),
                  pl.BlockSpec((DIM, DIM * DIM), lambda i: (0, 0)),
                  pl.BlockSpec((DIM * DIM, DIM), lambda i: (0, 0))],
        out_specs=pl.BlockSpec((T, W), lambda i: (i, 0)),
        out_shape=jax.ShapeDtypeStruct((E, W), jnp.float32),
    )(hid, xj, We2T, be2.reshape(1, DIM * DIM), Rrep, Ssum)


def _tc_mean_gru(s0, s1, h, WihT, WhhT, bih, bhh, bconv):
    def body(s0_ref, s1_ref, h_ref, wi_ref, wh_ref,
             bi_ref, bh_ref, bc_ref, o_ref):
        ssum = s0_ref[0:N, 0:DIM] + s1_ref[0:N, 0:DIM]
        cnt = s0_ref[0:N, DIM:DIM + 1] + s1_ref[0:N, DIM:DIM + 1]
        aggr = ssum / jnp.maximum(cnt, 1.0) + bc_ref[...]
        m = jnp.maximum(aggr, 0.0)
        h = h_ref[:, 0:DIM]
        gi = jnp.dot(m, wi_ref[...], preferred_element_type=jnp.float32) + bi_ref[...]
        gh = jnp.dot(h, wh_ref[...], preferred_element_type=jnp.float32) + bh_ref[...]
        r = _sigmoid(gi[:, 0:DIM] + gh[:, 0:DIM])
        z = _sigmoid(gi[:, DIM:2 * DIM] + gh[:, DIM:2 * DIM])
        n = jnp.tanh(gi[:, 2 * DIM:] + r * gh[:, 2 * DIM:])
        hn = (1.0 - z) * n + z * h
        o_ref[...] = jnp.concatenate(
            [hn, jnp.zeros((N, W - DIM), jnp.float32)], axis=1)

    return pl.pallas_call(
        body,
        out_shape=jax.ShapeDtypeStruct((N, W), jnp.float32),
    )(s0, s1, h, WihT, WhhT,
      bih.reshape(1, 3 * DIM), bhh.reshape(1, 3 * DIM), bconv.reshape(1, DIM))


def _tc_set2set(out, batch2, WsihT, WshhT, bsih, bshh):
    def body(out_ref, b_ref, wi_ref, wh_ref, bi_ref, bh_ref, q_ref, f_ref):
        feat = out_ref[:, 0:DIM]
        f_ref[...] = feat
        seg = b_ref[...]
        oh = (seg == lax.broadcasted_iota(jnp.int32, (N, B), 1)
              ).astype(jnp.float32)
        q_star = jnp.zeros((B, 2 * DIM), jnp.float32)
        hs = jnp.zeros((B, DIM), jnp.float32)
        cs = jnp.zeros((B, DIM), jnp.float32)
        dn = (((0,), (0,)), ((), ()))
        for _ in range(3):
            g = (jnp.dot(q_star, wi_ref[...], preferred_element_type=jnp.float32)
                 + bi_ref[...]
                 + jnp.dot(hs, wh_ref[...], preferred_element_type=jnp.float32)
                 + bh_ref[...])
            i = _sigmoid(g[:, 0:DIM])
            f = _sigmoid(g[:, DIM:2 * DIM])
            gg = jnp.tanh(g[:, 2 * DIM:3 * DIM])
            o = _sigmoid(g[:, 3 * DIM:])
            cs = f * cs + i * gg
            hs = o * jnp.tanh(cs)
            qn = jnp.dot(oh, hs, preferred_element_type=jnp.float32)
            e = jnp.sum(feat * qn, axis=1, keepdims=True)
            emax = jnp.max(jnp.where(oh > 0.0, e, -1e30), axis=0,
                           keepdims=True)
            emaxn = jnp.dot(oh, emax.reshape(B, 1),
                            preferred_element_type=jnp.float32)
            a = jnp.exp(e - emaxn)
            denom = lax.dot_general(oh, a, dn,
                                    preferred_element_type=jnp.float32)
            denomn = jnp.dot(oh, denom, preferred_element_type=jnp.float32)
            a = a / denomn
            r = lax.dot_general(oh, a * feat, dn,
                                preferred_element_type=jnp.float32)
            q_star = jnp.concatenate([hs, r], axis=1)
        q_ref[...] = q_star

    return pl.pallas_call(
        body,
        out_shape=[jax.ShapeDtypeStruct((B, 2 * DIM), jnp.float32),
                   jax.ShapeDtypeStruct((N, DIM), jnp.float32)],
    )(out, batch2, WsihT, WshhT,
      bsih.reshape(1, 4 * DIM), bshh.reshape(1, 4 * DIM))


def kernel(x, edge_index, edge_attr, batch, W0, b0, We1, be1, We2, be2,
           bconv, Wih, Whh, bih, bhh, Ws_ih, Ws_hh, bs_ih, bs_hh):
    src = edge_index[0]
    dst = edge_index[1]
    attrP = jnp.pad(edge_attr, ((0, 0), (0, 3)))
    We1TP = jnp.pad(We1.T, ((0, 3), (0, 0)))
    zrows = jnp.zeros((NPAD, W), jnp.float32)
    batch2 = batch.reshape(N, 1)

    out = _tc_proj(x, W0.T, b0)
    hid = _tc_hid(attrP, We1TP, be1)

    We2T = We2.T
    WihT = Wih.T
    WhhT = Whh.T
    Rrep = jnp.repeat(jnp.eye(DIM, dtype=jnp.float32), DIM, axis=1)
    Ssum = jnp.tile(jnp.eye(DIM, dtype=jnp.float32), (DIM, 1))
    for _ in range(3):
        xj = _sc_gather(out, src)
        msg = _tc_msg(hid, xj, We2T, be2, Rrep, Ssum)
        s0, s1 = _sc_scatter_sum(msg, dst, zrows)
        out = _tc_mean_gru(s0, s1, out, WihT, WhhT, bih, bhh, bconv)

    q_star, feat = _tc_set2set(out, batch2, Ws_ih.T, Ws_hh.T, bs_ih, bs_hh)
    return (q_star, feat)


# speedup vs baseline: 2.7619x; 1.0061x over previous
import functools

import jax
import jax.numpy as jnp
from jax import lax
from jax.experimental import pallas as pl
from jax.experimental.pallas import tpu as pltpu
from jax.experimental.pallas import tpu_sc as plsc

N = 10000
E = 160000
DIM = 32
B = 128
FIN = 128
W = 128
WS = 64

NC = 2
NS = 16
NW = NC * NS
PER_W = E // NW
CH = 200
NCHUNK = PER_W // CH
ZROWS = 632
NPAD = ZROWS * NS


def _sc_mesh():
    return dict(mesh=plsc.VectorSubcoreMesh(core_axis_name="c",
                                            subcore_axis_name="s",
                                            num_cores=NC, num_subcores=NS))


def _sc_gather(table, src):
    @functools.partial(
        pl.kernel,
        out_type=jax.ShapeDtypeStruct((E, W), jnp.float32),
        scratch_types=[pltpu.VMEM((PER_W,), jnp.int32),
                       pltpu.VMEM((CH, W), jnp.float32),
                       pltpu.VMEM((CH, W), jnp.float32),
                       pltpu.SemaphoreType.DMA,
                       pltpu.SemaphoreType.DMA,
                       pltpu.SemaphoreType.DMA,
                       pltpu.SemaphoreType.DMA],
        **_sc_mesh(),
    )
    def k(table_h, src_h, xj_h, idx_all, rows0, rows1, sg0, sg1, sw0, sw1):
        wid = lax.axis_index("c") * NS + lax.axis_index("s")
        base_w = pl.multiple_of(wid * PER_W, 8)
        pltpu.sync_copy(src_h.at[pl.ds(base_w, PER_W)], idx_all)
        rows = (rows0, rows1)
        sg = (sg0, sg1)
        sw = (sw0, sw1)
        hg = [None, None]
        hw = [None, None]
        for ci in range(NCHUNK):
            b = ci & 1
            if hw[b] is not None:
                hw[b].wait()
                hw[b] = None
            hg[b] = pltpu.async_copy(
                table_h.at[idx_all.at[pl.ds(ci * CH, CH)]], rows[b], sg[b])
            if ci > 0:
                pb = 1 - b
                hg[pb].wait()
                hw[pb] = pltpu.async_copy(
                    rows[pb],
                    xj_h.at[pl.ds(base_w + (ci - 1) * CH, CH)], sw[pb])
        bl = (NCHUNK - 1) & 1
        hg[bl].wait()
        hw[bl] = pltpu.async_copy(
            rows[bl], xj_h.at[pl.ds(base_w + (NCHUNK - 1) * CH, CH)], sw[bl])
        for b in range(2):
            if hw[b] is not None:
                hw[b].wait()

    return k(table, src)


def _sc_scatter_sum(msg, dst, zrows):
    @functools.partial(
        pl.kernel,
        out_type=[jax.ShapeDtypeStruct((NPAD, W), jnp.float32),
                  jax.ShapeDtypeStruct((NPAD, W), jnp.float32)],
        scratch_types=[pltpu.VMEM((CH,), jnp.int32),
                       pltpu.VMEM((CH, W), jnp.float32),
                       pltpu.VMEM_SHARED((NPAD, W), jnp.float32)],
        **_sc_mesh(),
    )
    def k(msg_h, dst_h, zeros_h, out0_h, out1_h, idx_v, rows_v, acc_s):
        c = lax.axis_index("c")
        s = lax.axis_index("s")
        wid = c * NS + s
        base_w = pl.multiple_of(wid * PER_W, 8)
        zb = pl.multiple_of(s * ZROWS, 8)
        pltpu.sync_copy(zeros_h.at[pl.ds(zb, ZROWS)], acc_s.at[pl.ds(zb, ZROWS)])
        plsc.subcore_barrier()

        def chunk(ci, carry):
            base = pl.multiple_of(base_w + ci * CH, 8)
            pltpu.sync_copy(dst_h.at[pl.ds(base, CH)], idx_v)
            pltpu.sync_copy(msg_h.at[pl.ds(base, CH)], rows_v)
            pltpu.sync_copy(rows_v, acc_s.at[idx_v], add=True)
            return carry

        lax.fori_loop(0, NCHUNK, chunk, 0)
        plsc.subcore_barrier()

        @pl.when(c == 0)
        def _():
            pltpu.sync_copy(acc_s.at[pl.ds(zb, ZROWS)], out0_h.at[pl.ds(zb, ZROWS)])

        @pl.when(c == 1)
        def _():
            pltpu.sync_copy(acc_s.at[pl.ds(zb, ZROWS)], out1_h.at[pl.ds(zb, ZROWS)])

    return k(msg, dst, zrows)


def _sigmoid(x):
    return 1.0 / (1.0 + jnp.exp(-x))


def _tc_proj(x, W0T, b0):
    T = 2000

    def body(x_ref, w_ref, b_ref, o_ref):
        h = jnp.maximum(
            jnp.dot(x_ref[...], w_ref[...],
                    preferred_element_type=jnp.float32) + b_ref[...], 0.0)
        o_ref[...] = jnp.concatenate(
            [h, jnp.zeros((T, W - DIM), jnp.float32)], axis=1)

    return pl.pallas_call(
        body,
        grid=(N // T,),
        in_specs=[pl.BlockSpec((T, FIN), lambda i: (i, 0)),
                  pl.BlockSpec((FIN, DIM), lambda i: (0, 0)),
                  pl.BlockSpec((1, DIM), lambda i: (0, 0))],
        out_specs=pl.BlockSpec((T, W), lambda i: (i, 0)),
        out_shape=jax.ShapeDtypeStruct((N, W), jnp.float32),
    )(x, W0T, b0.reshape(1, DIM))


def _tc_hid(attrP, We1TP, be1):
    T = 2000

    def body(a_ref, w_ref, b_ref, o_ref):
        o_ref[...] = jnp.maximum(
            jnp.dot(a_ref[...], w_ref[...],
                    preferred_element_type=jnp.float32) + b_ref[...],
            0.0).astype(jnp.bfloat16)

    return pl.pallas_call(
        body,
        grid=(E // T,),
        in_specs=[pl.BlockSpec((T, 8), lambda i: (i, 0)),
                  pl.BlockSpec((8, 128), lambda i: (0, 0)),
                  pl.BlockSpec((1, 128), lambda i: (0, 0))],
        out_specs=pl.BlockSpec((T, 128), lambda i: (i, 0)),
        out_shape=jax.ShapeDtypeStruct((E, 128), jnp.bfloat16),
    )(attrP, We1TP, be1.reshape(1, 128))


def _tc_msg(hid, xj, We2T, be2, Rrep, Ssum):
    T = 1600

    def body(h_ref, x_ref, w_ref, b_ref, r_ref, s_ref, o_ref):
        ew = jnp.dot(h_ref[...],
                     w_ref[...].astype(jnp.bfloat16),
                     preferred_element_type=jnp.float32) + b_ref[...]
        xjb = jnp.dot(x_ref[:, 0:DIM].astype(jnp.bfloat16),
                      r_ref[...].astype(jnp.bfloat16),
                      preferred_element_type=jnp.float32)
        acc = jnp.dot((xjb * ew).astype(jnp.bfloat16),
                      s_ref[...].astype(jnp.bfloat16),
                      preferred_element_type=jnp.float32)
        o_ref[...] = jnp.concatenate(
            [acc, jnp.ones((T, 1), jnp.float32),
             jnp.zeros((T, W - DIM - 1), jnp.float32)], axis=1)

    return pl.pallas_call(
        body,
        grid=(E // T,),
        in_specs=[pl.BlockSpec((T, 128), lambda i: (i, 0)),
                  pl.BlockSpec((T, W), lambda i: (i, 0)),
                  pl.BlockSpec((128, DIM * DIM), lambda i: (0, 0)),
                  pl.BlockSpec((1, DIM * DIM), lambda i: (0, 0)),
                  pl.BlockSpec((DIM, DIM * DIM), lambda i: (0, 0)),
                  pl.BlockSpec((DIM * DIM, DIM), lambda i: (0, 0))],
        out_specs=pl.BlockSpec((T, W), lambda i: (i, 0)),
        out_shape=jax.ShapeDtypeStruct((E, W), jnp.float32),
    )(hid, xj, We2T, be2.reshape(1, DIM * DIM), Rrep, Ssum)


def _tc_mean_gru(s0, s1, h, WihT, WhhT, bih, bhh, bconv):
    def body(s0_ref, s1_ref, h_ref, wi_ref, wh_ref,
             bi_ref, bh_ref, bc_ref, o_ref):
        ssum = s0_ref[0:N, 0:DIM] + s1_ref[0:N, 0:DIM]
        cnt = s0_ref[0:N, DIM:DIM + 1] + s1_ref[0:N, DIM:DIM + 1]
        aggr = ssum / jnp.maximum(cnt, 1.0) + bc_ref[...]
        m = jnp.maximum(aggr, 0.0)
        h = h_ref[:, 0:DIM]
        gi = jnp.dot(m, wi_ref[...], preferred_element_type=jnp.float32) + bi_ref[...]
        gh = jnp.dot(h, wh_ref[...], preferred_element_type=jnp.float32) + bh_ref[...]
        r = _sigmoid(gi[:, 0:DIM] + gh[:, 0:DIM])
        z = _sigmoid(gi[:, DIM:2 * DIM] + gh[:, DIM:2 * DIM])
        n = jnp.tanh(gi[:, 2 * DIM:] + r * gh[:, 2 * DIM:])
        hn = (1.0 - z) * n + z * h
        o_ref[...] = jnp.concatenate(
            [hn, jnp.zeros((N, W - DIM), jnp.float32)], axis=1)

    return pl.pallas_call(
        body,
        out_shape=jax.ShapeDtypeStruct((N, W), jnp.float32),
    )(s0, s1, h, WihT, WhhT,
      bih.reshape(1, 3 * DIM), bhh.reshape(1, 3 * DIM), bconv.reshape(1, DIM))


def _tc_set2set(out, batch2, WsihT, WshhT, bsih, bshh):
    def body(out_ref, b_ref, wi_ref, wh_ref, bi_ref, bh_ref, q_ref, f_ref):
        feat = out_ref[:, 0:DIM]
        f_ref[...] = feat
        seg = b_ref[...]
        oh = (seg == lax.broadcasted_iota(jnp.int32, (N, B), 1)
              ).astype(jnp.float32)
        q_star = jnp.zeros((B, 2 * DIM), jnp.float32)
        hs = jnp.zeros((B, DIM), jnp.float32)
        cs = jnp.zeros((B, DIM), jnp.float32)
        dn = (((0,), (0,)), ((), ()))
        for _ in range(3):
            g = (jnp.dot(q_star, wi_ref[...], preferred_element_type=jnp.float32)
                 + bi_ref[...]
                 + jnp.dot(hs, wh_ref[...], preferred_element_type=jnp.float32)
                 + bh_ref[...])
            i = _sigmoid(g[:, 0:DIM])
            f = _sigmoid(g[:, DIM:2 * DIM])
            gg = jnp.tanh(g[:, 2 * DIM:3 * DIM])
            o = _sigmoid(g[:, 3 * DIM:])
            cs = f * cs + i * gg
            hs = o * jnp.tanh(cs)
            qn = jnp.dot(oh, hs, preferred_element_type=jnp.float32)
            e = jnp.sum(feat * qn, axis=1, keepdims=True)
            emax = jnp.max(jnp.where(oh > 0.0, e, -1e30), axis=0,
                           keepdims=True)
            emaxn = jnp.dot(oh, emax.reshape(B, 1),
                            preferred_element_type=jnp.float32)
            a = jnp.exp(e - emaxn)
            denom = lax.dot_general(oh, a, dn,
                                    preferred_element_type=jnp.float32)
            denomn = jnp.dot(oh, denom, preferred_element_type=jnp.float32)
            a = a / denomn
            r = lax.dot_general(oh, a * feat, dn,
                                preferred_element_type=jnp.float32)
            q_star = jnp.concatenate([hs, r], axis=1)
        q_ref[...] = q_star

    return pl.pallas_call(
        body,
        out_shape=[jax.ShapeDtypeStruct((B, 2 * DIM), jnp.float32),
                   jax.ShapeDtypeStruct((N, DIM), jnp.float32)],
    )(out, batch2, WsihT, WshhT,
      bsih.reshape(1, 4 * DIM), bshh.reshape(1, 4 * DIM))


def kernel(x, edge_index, edge_attr, batch, W0, b0, We1, be1, We2, be2,
           bconv, Wih, Whh, bih, bhh, Ws_ih, Ws_hh, bs_ih, bs_hh):
    src = edge_index[0]
    dst = edge_index[1]
    attrP = jnp.pad(edge_attr, ((0, 0), (0, 3)))
    We1TP = jnp.pad(We1.T, ((0, 3), (0, 0)))
    zrows = jnp.zeros((NPAD, W), jnp.float32)
    batch2 = batch.reshape(N, 1)

    out = _tc_proj(x, W0.T, b0)
    hid = _tc_hid(attrP, We1TP, be1)

    We2T = We2.T
    WihT = Wih.T
    WhhT = Whh.T
    Rrep = jnp.repeat(jnp.eye(DIM, dtype=jnp.float32), DIM, axis=1)
    Ssum = jnp.tile(jnp.eye(DIM, dtype=jnp.float32), (DIM, 1))
    for _ in range(3):
        xj = _sc_gather(out, src)
        msg = _tc_msg(hid, xj, We2T, be2, Rrep, Ssum)
        s0, s1 = _sc_scatter_sum(msg, dst, zrows)
        out = _tc_mean_gru(s0, s1, out, WihT, WhhT, bih, bhh, bconv)

    q_star, feat = _tc_set2set(out, batch2, Ws_ih.T, Ws_hh.T, bs_ih, bs_hh)
    return (q_star, feat)
